# baseline (speedup 1.0000x reference)
import functools

import jax
import jax.numpy as jnp
from jax import lax
from jax.experimental import pallas as pl
from jax.experimental.pallas import tpu as pltpu
from jax.experimental.pallas import tpu_sc as plsc

BATCH = 16384
EMB = 64
NC = 2
NS = 16
NW = NC * NS
BPW = BATCH // NW


def _gather_body(uid_hbm, iid_hbm, utab_hbm, itab_hbm, u_out, v_out,
                 uidx_v, iidx_v, urows_v, irows_v, usem, isem):
    wid = lax.axis_index("s") * NC + lax.axis_index("c")
    base = wid * BPW
    pltpu.sync_copy(uid_hbm.at[pl.ds(base, BPW)], uidx_v)
    pltpu.sync_copy(iid_hbm.at[pl.ds(base, BPW)], iidx_v)
    cu = pltpu.async_copy(utab_hbm.at[uidx_v], urows_v, usem)
    ci = pltpu.async_copy(itab_hbm.at[iidx_v], irows_v, isem)
    cu.wait()
    ci.wait()
    pltpu.sync_copy(urows_v, u_out.at[pl.ds(base, BPW)])
    pltpu.sync_copy(irows_v, v_out.at[pl.ds(base, BPW)])


@jax.jit
def _gather(user_ids, item_ids, user_table, item_table):
    mesh = plsc.VectorSubcoreMesh(core_axis_name="c", subcore_axis_name="s")
    f = pl.kernel(
        _gather_body,
        out_type=(
            jax.ShapeDtypeStruct((BATCH, EMB), jnp.float32),
            jax.ShapeDtypeStruct((BATCH, EMB), jnp.float32),
        ),
        mesh=mesh,
        scratch_types=[
            pltpu.VMEM((BPW,), jnp.int32),
            pltpu.VMEM((BPW,), jnp.int32),
            pltpu.VMEM((BPW, EMB), jnp.float32),
            pltpu.VMEM((BPW, EMB), jnp.float32),
            pltpu.SemaphoreType.DMA,
            pltpu.SemaphoreType.DMA,
        ],
        compiler_params=pltpu.CompilerParams(use_tc_tiling_on_sc=False),
    )
    return f(user_ids, item_ids, user_table, item_table)


def _transpose_body(t_ref, o_ref):
    o_ref[...] = t_ref[...].T


@jax.jit
def _transpose(tT):
    bs = 2048
    n = tT.shape[1]
    grid = (pl.cdiv(n, bs),)
    return pl.pallas_call(
        _transpose_body,
        grid=grid,
        in_specs=[pl.BlockSpec((EMB, bs), lambda i: (0, i))],
        out_specs=pl.BlockSpec((bs, EMB), lambda i: (i, 0)),
        out_shape=jax.ShapeDtypeStruct((n, EMB), jnp.float32),
    )(tT)


def _mlp_body(u_ref, v_ref, w0u_ref, w0v_ref, b0_ref, w1_ref, b1_ref,
              w2_ref, b2_ref, wout_ref, bout_ref, o_ref):
    x = u_ref[...] @ w0u_ref[...] + v_ref[...] @ w0v_ref[...] + b0_ref[...]
    x = jnp.maximum(x, 0.0)
    x = jnp.maximum(x @ w1_ref[...] + b1_ref[...], 0.0)
    x = jnp.maximum(x @ w2_ref[...] + b2_ref[...], 0.0)
    o_ref[...] = jax.nn.sigmoid(x @ wout_ref[...] + bout_ref[...])


@functools.partial(jax.jit, static_argnames=("bs",))
def _mlp(u, v, w0u, w0v, b0, w1, b1, w2, b2, wout, bout, bs=2048):
    grid = (BATCH // bs,)
    full = lambda shape: pl.BlockSpec(shape, lambda i: (0, 0))
    return pl.pallas_call(
        _mlp_body,
        grid=grid,
        in_specs=[
            pl.BlockSpec((bs, EMB), lambda i: (i, 0)),
            pl.BlockSpec((bs, EMB), lambda i: (i, 0)),
            full((EMB, 128)),
            full((EMB, 128)),
            full((1, 128)),
            full((128, 64)),
            full((1, 64)),
            full((64, 32)),
            full((1, 32)),
            full((32, 1)),
            full((1, 1)),
        ],
        out_specs=pl.BlockSpec((bs, 1), lambda i: (i, 0)),
        out_shape=jax.ShapeDtypeStruct((BATCH, 1), jnp.float32),
    )(u, v, w0u, w0v, b0, w1, b1, w2, b2, wout, bout)


def kernel(user_ids, item_ids, user_table, item_table,
           W0, b0, W1, b1, W2, b2, Wout, bout):
    ut = _transpose(user_table.T)
    it = _transpose(item_table.T)
    u, v = _gather(user_ids.astype(jnp.int32), item_ids.astype(jnp.int32),
                   ut, it)
    out = _mlp(u, v, W0[:EMB], W0[EMB:], b0.reshape(1, -1),
               W1, b1.reshape(1, -1), W2, b2.reshape(1, -1),
               Wout, bout.reshape(1, 1))
    return out.reshape(BATCH)


# speedup vs baseline: 1.3279x; 1.3279x over previous
import functools

import jax
import jax.numpy as jnp
from jax import lax
from jax.experimental import pallas as pl
from jax.experimental.pallas import tpu as pltpu
from jax.experimental.pallas import tpu_sc as plsc

BATCH = 16384
EMB = 64
NC = 2
NS = 16
NW = NC * NS
BPW = BATCH // NW


def _gather_body(uid_hbm, iid_hbm, utab_hbm, itab_hbm, u_out, v_out,
                 uidx_v, iidx_v, urows_v, irows_v, usem, isem):
    wid = lax.axis_index("s") * NC + lax.axis_index("c")
    base = wid * BPW
    pltpu.sync_copy(uid_hbm.at[pl.ds(base, BPW)], uidx_v)
    pltpu.sync_copy(iid_hbm.at[pl.ds(base, BPW)], iidx_v)
    cu = pltpu.async_copy(utab_hbm.at[uidx_v], urows_v, usem)
    ci = pltpu.async_copy(itab_hbm.at[iidx_v], irows_v, isem)
    cu.wait()
    ci.wait()
    pltpu.sync_copy(urows_v, u_out.at[pl.ds(base, BPW)])
    pltpu.sync_copy(irows_v, v_out.at[pl.ds(base, BPW)])


@jax.jit
def _gather(user_ids, item_ids, user_table, item_table):
    mesh = plsc.VectorSubcoreMesh(core_axis_name="c", subcore_axis_name="s")
    f = pl.kernel(
        _gather_body,
        out_type=(
            jax.ShapeDtypeStruct((BATCH, EMB), jnp.float32),
            jax.ShapeDtypeStruct((BATCH, EMB), jnp.float32),
        ),
        mesh=mesh,
        scratch_types=[
            pltpu.VMEM((BPW,), jnp.int32),
            pltpu.VMEM((BPW,), jnp.int32),
            pltpu.VMEM((BPW, EMB), jnp.float32),
            pltpu.VMEM((BPW, EMB), jnp.float32),
            pltpu.SemaphoreType.DMA,
            pltpu.SemaphoreType.DMA,
        ],
        compiler_params=pltpu.CompilerParams(use_tc_tiling_on_sc=False),
    )
    return f(user_ids, item_ids, user_table, item_table)


def _transpose_body(t_ref, o_ref):
    o_ref[...] = t_ref[...].T


@jax.jit
def _transpose(tT):
    bs = 16384
    n = tT.shape[1]
    grid = (pl.cdiv(n, bs),)
    return pl.pallas_call(
        _transpose_body,
        grid=grid,
        in_specs=[pl.BlockSpec((EMB, bs), lambda i: (0, i))],
        out_specs=pl.BlockSpec((bs, EMB), lambda i: (i, 0)),
        out_shape=jax.ShapeDtypeStruct((n, EMB), jnp.float32),
    )(tT)


def _mlp_body(u_ref, v_ref, w0u_ref, w0v_ref, b0_ref, w1_ref, b1_ref,
              w2_ref, b2_ref, wout_ref, bout_ref, o_ref):
    x = u_ref[...] @ w0u_ref[...] + v_ref[...] @ w0v_ref[...] + b0_ref[...]
    x = jnp.maximum(x, 0.0)
    x = jnp.maximum(x @ w1_ref[...] + b1_ref[...], 0.0)
    x = jnp.maximum(x @ w2_ref[...] + b2_ref[...], 0.0)
    o_ref[...] = jax.nn.sigmoid(x @ wout_ref[...] + bout_ref[...])


@functools.partial(jax.jit, static_argnames=("bs",))
def _mlp(u, v, w0u, w0v, b0, w1, b1, w2, b2, wout, bout, bs=2048):
    grid = (BATCH // bs,)
    full = lambda shape: pl.BlockSpec(shape, lambda i: (0, 0))
    return pl.pallas_call(
        _mlp_body,
        grid=grid,
        in_specs=[
            pl.BlockSpec((bs, EMB), lambda i: (i, 0)),
            pl.BlockSpec((bs, EMB), lambda i: (i, 0)),
            full((EMB, 128)),
            full((EMB, 128)),
            full((1, 128)),
            full((128, 64)),
            full((1, 64)),
            full((64, 32)),
            full((1, 32)),
            full((32, 1)),
            full((1, 1)),
        ],
        out_specs=pl.BlockSpec((bs, 1), lambda i: (i, 0)),
        out_shape=jax.ShapeDtypeStruct((BATCH, 1), jnp.float32),
    )(u, v, w0u, w0v, b0, w1, b1, w2, b2, wout, bout)


def kernel(user_ids, item_ids, user_table, item_table,
           W0, b0, W1, b1, W2, b2, Wout, bout):
    ut = _transpose(user_table.T)
    it = _transpose(item_table.T)
    u, v = _gather(user_ids.astype(jnp.int32), item_ids.astype(jnp.int32),
                   ut, it)
    out = _mlp(u, v, W0[:EMB], W0[EMB:], b0.reshape(1, -1),
               W1, b1.reshape(1, -1), W2, b2.reshape(1, -1),
               Wout, bout.reshape(1, 1))
    return out.reshape(BATCH)


# speedup vs baseline: 1.5320x; 1.1537x over previous
import functools

import jax
import jax.numpy as jnp
from jax import lax
from jax.experimental import pallas as pl
from jax.experimental.pallas import tpu as pltpu
from jax.experimental.pallas import tpu_sc as plsc

BATCH = 16384
EMB = 64
NROWS2 = 500000
NC = 2
NS = 16
NW = NC * NS
BPW = BATCH // NW


def _gather2_body(uid_hbm, iid_hbm, tu_hbm, tv_hbm, uo_hbm, vo_hbm,
                  uidx_v, iidx_v, rows_v, sem):
    wid = lax.axis_index("s") * NC + lax.axis_index("c")
    base = wid * BPW
    pltpu.sync_copy(uid_hbm.at[pl.ds(base, BPW)], uidx_v)
    pltpu.sync_copy(iid_hbm.at[pl.ds(base, BPW)], iidx_v)
    pltpu.async_copy(tu_hbm.at[uidx_v], rows_v, sem).wait()
    pltpu.sync_copy(rows_v, uo_hbm.at[pl.ds(base, BPW)])
    pltpu.async_copy(tv_hbm.at[iidx_v], rows_v, sem).wait()
    pltpu.sync_copy(rows_v, vo_hbm.at[pl.ds(base, BPW)])


@jax.jit
def _gather2(uhid, ihid, tu, tv):
    mesh = plsc.VectorSubcoreMesh(core_axis_name="c", subcore_axis_name="s")
    f = pl.kernel(
        _gather2_body,
        out_type=(
            jax.ShapeDtypeStruct((BATCH, 128), jnp.float32),
            jax.ShapeDtypeStruct((BATCH, 128), jnp.float32),
        ),
        mesh=mesh,
        scratch_types=[
            pltpu.VMEM((BPW,), jnp.int32),
            pltpu.VMEM((BPW,), jnp.int32),
            pltpu.VMEM((BPW, 128), jnp.float32),
            pltpu.SemaphoreType.DMA,
        ],
        compiler_params=pltpu.CompilerParams(use_tc_tiling_on_sc=True),
    )
    return f(uhid, ihid, tu, tv)


def _mlp_body(up_ref, vp_ref, pu_ref, pv_ref, w0u_ref, w0v_ref, b0_ref,
              w1_ref, b1_ref, w2_ref, b2_ref, wout_ref, bout_ref, o_ref):
    up = up_ref[...]
    vp = vp_ref[...]
    u = jnp.where(pu_ref[...] > 0.5, up[:, EMB:], up[:, :EMB])
    v = jnp.where(pv_ref[...] > 0.5, vp[:, EMB:], vp[:, :EMB])
    x = u @ w0u_ref[...] + v @ w0v_ref[...] + b0_ref[...]
    x = jnp.maximum(x, 0.0)
    x = jnp.maximum(x @ w1_ref[...] + b1_ref[...], 0.0)
    x = jnp.maximum(x @ w2_ref[...] + b2_ref[...], 0.0)
    o_ref[...] = jax.nn.sigmoid(x @ wout_ref[...] + bout_ref[...])


@functools.partial(jax.jit, static_argnames=("bs",))
def _mlp(up, vp, pu, pv, w0u, w0v, b0, w1, b1, w2, b2, wout, bout, bs=2048):
    grid = (BATCH // bs,)
    full = lambda shape: pl.BlockSpec(shape, lambda i: (0, 0))
    return pl.pallas_call(
        _mlp_body,
        grid=grid,
        in_specs=[
            pl.BlockSpec((bs, 128), lambda i: (i, 0)),
            pl.BlockSpec((bs, 128), lambda i: (i, 0)),
            pl.BlockSpec((bs, 1), lambda i: (i, 0)),
            pl.BlockSpec((bs, 1), lambda i: (i, 0)),
            full((EMB, 128)),
            full((EMB, 128)),
            full((1, 128)),
            full((128, 64)),
            full((1, 64)),
            full((64, 32)),
            full((1, 32)),
            full((32, 1)),
            full((1, 1)),
        ],
        out_specs=pl.BlockSpec((bs, 1), lambda i: (i, 0)),
        out_shape=jax.ShapeDtypeStruct((BATCH, 1), jnp.float32),
    )(up, vp, pu, pv, w0u, w0v, b0, w1, b1, w2, b2, wout, bout)


def kernel(user_ids, item_ids, user_table, item_table,
           W0, b0, W1, b1, W2, b2, Wout, bout):
    uid = user_ids.astype(jnp.int32)
    iid = item_ids.astype(jnp.int32)
    tu = user_table.reshape(NROWS2, 128)
    tv = item_table.reshape(NROWS2, 128)
    up, vp = _gather2(uid >> 1, iid >> 1, tu, tv)
    pu = (uid & 1).astype(jnp.float32).reshape(-1, 1)
    pv = (iid & 1).astype(jnp.float32).reshape(-1, 1)
    out = _mlp(up, vp, pu, pv, W0[:EMB], W0[EMB:], b0.reshape(1, -1),
               W1, b1.reshape(1, -1), W2, b2.reshape(1, -1),
               Wout, bout.reshape(1, 1))
    return out.reshape(BATCH)


# speedup vs baseline: 2.4597x; 1.6056x over previous
import functools

import jax
import jax.numpy as jnp
from jax import lax
from jax.experimental import pallas as pl
from jax.experimental.pallas import tpu as pltpu
from jax.experimental.pallas import tpu_sc as plsc

BATCH = 16384
EMB = 64
HSPLIT = 499712
TBS = 2048
NROWS2 = 245 * TBS
NC = 2
NS = 16
NW = NC * NS
BPW = BATCH // NW


def _transpose2_body(a_ref, b_ref, o_ref):
    o_ref[...] = jnp.concatenate([a_ref[...].T, b_ref[...].T], axis=1)


@jax.jit
def _transpose2(tT):
    grid = (NROWS2 // TBS,)
    return pl.pallas_call(
        _transpose2_body,
        grid=grid,
        in_specs=[
            pl.BlockSpec((EMB, TBS), lambda i: (0, i)),
            pl.BlockSpec((EMB, TBS), lambda i: (0, i + HSPLIT // TBS)),
        ],
        out_specs=pl.BlockSpec((TBS, 128), lambda i: (i, 0)),
        out_shape=jax.ShapeDtypeStruct((NROWS2, 128), jnp.float32),
    )(tT, tT)


def _gather2_body(uid_hbm, iid_hbm, tu_hbm, tv_hbm, uo_hbm, vo_hbm,
                  uidx_v, iidx_v, rows_v, sem):
    wid = lax.axis_index("s") * NC + lax.axis_index("c")
    base = wid * BPW
    pltpu.sync_copy(uid_hbm.at[pl.ds(base, BPW)], uidx_v)
    pltpu.sync_copy(iid_hbm.at[pl.ds(base, BPW)], iidx_v)
    pltpu.async_copy(tu_hbm.at[uidx_v], rows_v, sem).wait()
    pltpu.sync_copy(rows_v, uo_hbm.at[pl.ds(base, BPW)])
    pltpu.async_copy(tv_hbm.at[iidx_v], rows_v, sem).wait()
    pltpu.sync_copy(rows_v, vo_hbm.at[pl.ds(base, BPW)])


@jax.jit
def _gather2(uhid, ihid, tu, tv):
    mesh = plsc.VectorSubcoreMesh(core_axis_name="c", subcore_axis_name="s")
    f = pl.kernel(
        _gather2_body,
        out_type=(
            jax.ShapeDtypeStruct((BATCH, 128), jnp.float32),
            jax.ShapeDtypeStruct((BATCH, 128), jnp.float32),
        ),
        mesh=mesh,
        scratch_types=[
            pltpu.VMEM((BPW,), jnp.int32),
            pltpu.VMEM((BPW,), jnp.int32),
            pltpu.VMEM((BPW, 128), jnp.float32),
            pltpu.SemaphoreType.DMA,
        ],
        compiler_params=pltpu.CompilerParams(use_tc_tiling_on_sc=True),
    )
    return f(uhid, ihid, tu, tv)


def _mlp_body(up_ref, vp_ref, pu_ref, pv_ref, w0u_ref, w0v_ref, b0_ref,
              w1_ref, b1_ref, w2_ref, b2_ref, wout_ref, bout_ref, o_ref):
    up = up_ref[...]
    vp = vp_ref[...]
    u = jnp.where(pu_ref[...] > 0.5, up[:, EMB:], up[:, :EMB])
    v = jnp.where(pv_ref[...] > 0.5, vp[:, EMB:], vp[:, :EMB])
    x = u @ w0u_ref[...] + v @ w0v_ref[...] + b0_ref[...]
    x = jnp.maximum(x, 0.0)
    x = jnp.maximum(x @ w1_ref[...] + b1_ref[...], 0.0)
    x = jnp.maximum(x @ w2_ref[...] + b2_ref[...], 0.0)
    o_ref[...] = jax.nn.sigmoid(x @ wout_ref[...] + bout_ref[...])


@functools.partial(jax.jit, static_argnames=("bs",))
def _mlp(up, vp, pu, pv, w0u, w0v, b0, w1, b1, w2, b2, wout, bout, bs=2048):
    grid = (BATCH // bs,)
    full = lambda shape: pl.BlockSpec(shape, lambda i: (0, 0))
    return pl.pallas_call(
        _mlp_body,
        grid=grid,
        in_specs=[
            pl.BlockSpec((bs, 128), lambda i: (i, 0)),
            pl.BlockSpec((bs, 128), lambda i: (i, 0)),
            pl.BlockSpec((bs, 1), lambda i: (i, 0)),
            pl.BlockSpec((bs, 1), lambda i: (i, 0)),
            full((EMB, 128)),
            full((EMB, 128)),
            full((1, 128)),
            full((128, 64)),
            full((1, 64)),
            full((64, 32)),
            full((1, 32)),
            full((32, 1)),
            full((1, 1)),
        ],
        out_specs=pl.BlockSpec((bs, 1), lambda i: (i, 0)),
        out_shape=jax.ShapeDtypeStruct((BATCH, 1), jnp.float32),
    )(up, vp, pu, pv, w0u, w0v, b0, w1, b1, w2, b2, wout, bout)


def kernel(user_ids, item_ids, user_table, item_table,
           W0, b0, W1, b1, W2, b2, Wout, bout):
    uid = user_ids.astype(jnp.int32)
    iid = item_ids.astype(jnp.int32)
    tu = _transpose2(user_table.T)
    tv = _transpose2(item_table.T)
    uhid = jnp.where(uid < HSPLIT, uid, uid - HSPLIT)
    ihid = jnp.where(iid < HSPLIT, iid, iid - HSPLIT)
    up, vp = _gather2(uhid, ihid, tu, tv)
    pu = (uid >= HSPLIT).astype(jnp.float32).reshape(-1, 1)
    pv = (iid >= HSPLIT).astype(jnp.float32).reshape(-1, 1)
    out = _mlp(up, vp, pu, pv, W0[:EMB], W0[EMB:], b0.reshape(1, -1),
               W1, b1.reshape(1, -1), W2, b2.reshape(1, -1),
               Wout, bout.reshape(1, 1))
    return out.reshape(BATCH)
